# baseline (speedup 1.0000x reference)
import functools

import jax
import jax.numpy as jnp
from jax import lax
from jax.experimental import pallas as pl
from jax.experimental.pallas import tpu as pltpu
from jax.experimental.pallas import tpu_sc as plsc

NUM_BOXES = 100000
DIM = 64
ROWS = 2 * DIM
OUT_CHUNK = 4096


@functools.lru_cache(maxsize=None)
def _build(batch: int):
    info = plsc.get_sparse_core_info()
    nw = info.num_cores * info.num_subcores
    rows_per_w = ROWS // nw
    n_chunks = batch // OUT_CHUNK

    mesh = plsc.VectorSubcoreMesh(core_axis_name="c", subcore_axis_name="s")

    @functools.partial(
        pl.kernel,
        out_type=jax.ShapeDtypeStruct((ROWS, batch), jnp.float32),
        mesh=mesh,
        scratch_types=[
            pltpu.VMEM((batch,), jnp.int32),
            pltpu.VMEM((NUM_BOXES,), jnp.float32),
            pltpu.VMEM((OUT_CHUNK,), jnp.float32),
            pltpu.VMEM((OUT_CHUNK,), jnp.float32),
            pltpu.SemaphoreType.DMA,
            pltpu.SemaphoreType.DMA,
            pltpu.SemaphoreType.DMA,
            pltpu.SemaphoreType.DMA,
        ],
        compiler_params=pltpu.CompilerParams(
            use_tc_tiling_on_sc=True,
            needs_layout_passes=False,
            disable_bounds_checks=True,
        ),
    )
    def gather_kernel(
        tbl_hbm, idx_hbm, out_hbm, idx_v, row_v, ob0_v, ob1_v, sem_i, sem_r, sem_o0, sem_o1
    ):
        obufs = (ob0_v, ob1_v)
        osems = (sem_o0, sem_o1)
        wid = lax.axis_index("s") * info.num_cores + lax.axis_index("c")
        idx_cp = pltpu.make_async_copy(idx_hbm, idx_v, sem_i)
        idx_cp.start()
        out_cps = []
        for rl in range(rows_per_w):
            r = wid * rows_per_w + rl
            row_cp = pltpu.make_async_copy(tbl_hbm.at[r], row_v, sem_r)
            row_cp.start()
            row_cp.wait()
            if rl == 0:
                idx_cp.wait()
            for ch in range(n_chunks):
                g = rl * n_chunks + ch
                if len(out_cps) >= 2:
                    out_cps[g - 2].wait()
                buf = obufs[g % 2]
                def _gather(i, _ch=ch, _buf=buf):
                    ids = idx_v[pl.ds(_ch * OUT_CHUNK + i, 16)]
                    _buf[pl.ds(i, 16)] = plsc.load_gather(row_v, [ids])
                plsc.parallel_loop(0, OUT_CHUNK, 16, unroll=8)(_gather)
                cp = pltpu.make_async_copy(
                    buf, out_hbm.at[r, pl.ds(ch * OUT_CHUNK, OUT_CHUNK)], osems[g % 2]
                )
                cp.start()
                out_cps.append(cp)
        for cp in out_cps[-2:]:
            cp.wait()

    return gather_kernel


def kernel(box_indices, boxes):
    num_models, num_boxes, two, dim = boxes.shape
    batch = box_indices.shape[0]
    tbl_t = jnp.transpose(boxes, (0, 2, 3, 1)).reshape(two * dim, num_boxes)
    out_t = _build(batch)(tbl_t, box_indices.astype(jnp.int32))
    return out_t.reshape(num_models, two, dim, batch).transpose(0, 3, 1, 2)


# speedup vs baseline: 1.0371x; 1.0371x over previous
import functools

import jax
import jax.numpy as jnp
from jax import lax
from jax.experimental import pallas as pl
from jax.experimental.pallas import tpu as pltpu
from jax.experimental.pallas import tpu_sc as plsc

NUM_BOXES = 100000
DIM = 64
ROWS = 2 * DIM
OUT_CHUNK = 4096


@functools.lru_cache(maxsize=None)
def _build(batch: int):
    info = plsc.get_sparse_core_info()
    nw = info.num_cores * info.num_subcores
    rows_per_w = ROWS // nw
    n_chunks = batch // OUT_CHUNK

    mesh = plsc.VectorSubcoreMesh(core_axis_name="c", subcore_axis_name="s")

    @functools.partial(
        pl.kernel,
        out_type=jax.ShapeDtypeStruct((ROWS, batch), jnp.float32),
        mesh=mesh,
        scratch_types=[
            pltpu.VMEM_SHARED((batch,), jnp.int32),
            pltpu.VMEM((batch,), jnp.int32),
            pltpu.VMEM((NUM_BOXES,), jnp.float32),
            pltpu.VMEM((OUT_CHUNK,), jnp.float32),
            pltpu.VMEM((OUT_CHUNK,), jnp.float32),
            pltpu.SemaphoreType.DMA,
            pltpu.SemaphoreType.DMA,
            pltpu.SemaphoreType.DMA,
            pltpu.SemaphoreType.DMA,
        ],
        compiler_params=pltpu.CompilerParams(
            use_tc_tiling_on_sc=True,
            needs_layout_passes=False,
            disable_bounds_checks=True,
        ),
    )
    def gather_kernel(
        tbl_hbm, idx_hbm, out_hbm, idx_s, idx_v, row_v, ob0_v, ob1_v,
        sem_i, sem_r, sem_o0, sem_o1
    ):
        obufs = (ob0_v, ob1_v)
        osems = (sem_o0, sem_o1)
        sid = lax.axis_index("s")
        wid = sid * info.num_cores + lax.axis_index("c")
        @pl.when(sid == 0)
        def _stage_idx():
            pltpu.sync_copy(idx_hbm, idx_s)
        plsc.subcore_barrier()
        idx_cp = pltpu.make_async_copy(idx_s, idx_v, sem_i)
        idx_cp.start()
        out_cps = []
        for rl in range(rows_per_w):
            r = wid * rows_per_w + rl
            row_cp = pltpu.make_async_copy(tbl_hbm.at[r], row_v, sem_r)
            row_cp.start()
            row_cp.wait()
            if rl == 0:
                idx_cp.wait()
            for ch in range(n_chunks):
                g = rl * n_chunks + ch
                if len(out_cps) >= 2:
                    out_cps[g - 2].wait()
                buf = obufs[g % 2]
                def _gather(i, _ch=ch, _buf=buf):
                    ids = idx_v[pl.ds(_ch * OUT_CHUNK + i, 16)]
                    _buf[pl.ds(i, 16)] = plsc.load_gather(row_v, [ids])
                plsc.parallel_loop(0, OUT_CHUNK, 16, unroll=8)(_gather)
                cp = pltpu.make_async_copy(
                    buf, out_hbm.at[r, pl.ds(ch * OUT_CHUNK, OUT_CHUNK)], osems[g % 2]
                )
                cp.start()
                out_cps.append(cp)
        for cp in out_cps[-2:]:
            cp.wait()

    return gather_kernel


def kernel(box_indices, boxes):
    num_models, num_boxes, two, dim = boxes.shape
    batch = box_indices.shape[0]
    tbl_t = jnp.transpose(boxes, (0, 2, 3, 1)).reshape(two * dim, num_boxes)
    out_t = _build(batch)(tbl_t, box_indices.astype(jnp.int32))
    return out_t.reshape(num_models, two, dim, batch).transpose(0, 3, 1, 2)
